# baseline (speedup 1.0000x reference)
import jax
import jax.numpy as jnp
from jax import lax
from jax.experimental import pallas as pl
from jax.experimental.pallas import tpu as pltpu
from jax.experimental.pallas import tpu_sc as plsc

N, C = 16384, 1000
NT = 11264
BLOCK = 512
NW = 32
SC_ROWS = N - NT
ROWS_PER_W = SC_ROWS // NW
G = 16
NGROUPS = ROWS_PER_W // G
NFULL = 62
NEG_INF = float("-inf")


def _sc_body(s_hbm, t_hbm, a_hbm, b_hbm, t_buf, s_buf, a_stage, b_stage):
    wid = lax.axis_index("s") * 2 + lax.axis_index("c")
    iota = lax.broadcasted_iota(jnp.int32, (16,), 0)
    tail_valid = iota >= 8
    zero16 = jnp.zeros((16,), jnp.float32)
    ninf16 = jnp.full((16,), NEG_INF, jnp.float32)

    def group_body(g, carry):
        out0 = wid * ROWS_PER_W + g * G
        row0 = NT + out0
        pltpu.sync_copy(t_hbm.at[pl.ds(row0, G)], t_buf)
        pltpu.sync_copy(s_hbm.at[pl.ds(row0, G)], s_buf)

        def row_body(r, vecs):
            vec_a, vec_b = vecs

            @plsc.parallel_loop(0, NFULL, unroll=4,
                                carry=(ninf16, ninf16, ninf16))
            def pA(c, acc):
                vm1, vm2, vmS = acc
                tc = t_buf[r, pl.ds(c * 16, 16)]
                sc = s_buf[r, pl.ds(c * 16, 16)]
                hi = jnp.maximum(vm1, tc)
                lo = jnp.minimum(vm1, tc)
                return (hi, jnp.maximum(vm2, lo), jnp.maximum(vmS, sc))

            vm1, vm2, vmS = pA
            t62 = jnp.where(tail_valid, t_buf[r, pl.ds(984, 16)], NEG_INF)
            s62 = jnp.where(tail_valid, s_buf[r, pl.ds(984, 16)], NEG_INF)
            hi = jnp.maximum(vm1, t62)
            vm2 = jnp.maximum(vm2, jnp.minimum(vm1, t62))
            vm1 = hi
            vmS = jnp.maximum(vmS, s62)

            m1 = jnp.max(vm1)
            mS = jnp.max(vmS)
            eql = vm1 == m1
            nl = jnp.sum(jnp.where(eql, 1.0, 0.0))
            m2l = jnp.max(jnp.where(eql, NEG_INF, vm1))
            m2 = jnp.where(nl > 1.5, m1, jnp.maximum(m2l, jnp.max(vm2)))
            gap = m1 - m2
            sharp = (gap > 0.6) & (gap <= 0.8)
            scale = jnp.where(sharp, jnp.float32(1.0 / 0.7), jnp.float32(1.0))
            mTs = m1 * scale

            @plsc.parallel_loop(0, NFULL, unroll=4,
                                carry=(zero16, zero16, zero16))
            def pB(c, acc):
                veT, vnum, veS = acc
                tc = t_buf[r, pl.ds(c * 16, 16)]
                sc = s_buf[r, pl.ds(c * 16, 16)]
                ts = tc * scale
                e = jnp.exp(ts - mTs)
                return (veT + e, vnum + e * (ts - sc), veS + jnp.exp(sc - mS))

            veT, vnum, veS = pB
            ts62 = t62 * scale
            e62 = jnp.where(tail_valid, jnp.exp(ts62 - mTs), 0.0)
            seT = jnp.sum(veT + e62)
            num = jnp.sum(vnum + jnp.where(tail_valid, e62 * (ts62 - s62), 0.0))
            seS = jnp.sum(veS + jnp.where(tail_valid, jnp.exp(s62 - mS), 0.0))

            seT16 = seT + zero16
            a_cand = (num + zero16) / seT16 - mTs + mS
            b_cand = (seS + zero16) / seT16
            return (jnp.where(iota == r, a_cand, vec_a),
                    jnp.where(iota == r, b_cand, vec_b))

        vec_a, vec_b = lax.fori_loop(0, G, row_body, (zero16, zero16))
        a_stage[...] = vec_a
        b_stage[...] = vec_b
        pltpu.sync_copy(a_stage, a_hbm.at[pl.ds(out0, G)])
        pltpu.sync_copy(b_stage, b_hbm.at[pl.ds(out0, G)])
        return carry

    lax.fori_loop(0, NGROUPS, group_body, 0)


def _sc_partials(preds_S, preds_T):
    mesh = plsc.VectorSubcoreMesh(core_axis_name="c", subcore_axis_name="s")
    return pl.kernel(
        _sc_body,
        out_type=(
            jax.ShapeDtypeStruct((SC_ROWS,), jnp.float32),
            jax.ShapeDtypeStruct((SC_ROWS,), jnp.float32),
        ),
        mesh=mesh,
        compiler_params=pltpu.CompilerParams(needs_layout_passes=False),
        cost_estimate=pl.CostEstimate(
            flops=SC_ROWS * C * 12,
            transcendentals=SC_ROWS * C * 2,
            bytes_accessed=SC_ROWS * C * 8,
        ),
        scratch_types=[
            pltpu.VMEM((G, C), jnp.float32),
            pltpu.VMEM((G, C), jnp.float32),
            pltpu.VMEM((16,), jnp.float32),
            pltpu.VMEM((16,), jnp.float32),
        ],
    )(preds_S, preds_T)


def _kl_block(s_ref, t_ref, out_ref):
    t = t_ref[...]
    s = s_ref[...]

    m1 = jnp.max(t, axis=1, keepdims=True)
    eq = t == m1
    dup = jnp.sum(eq.astype(jnp.float32), axis=1, keepdims=True) > 1.5
    m2 = jnp.max(jnp.where(eq, -jnp.inf, t), axis=1, keepdims=True)
    gap = m1 - m2
    sharp = (gap > 0.6) & (gap <= 0.8) & jnp.logical_not(dup)
    scale = jnp.where(sharp, 1.0 / 0.7, 1.0)

    ts = t * scale
    mT = m1 * scale
    eT = jnp.exp(ts - mT)
    seT = jnp.sum(eT, axis=1, keepdims=True)
    num = jnp.sum(eT * (ts - s), axis=1, keepdims=True)

    mS = jnp.max(s, axis=1, keepdims=True)
    seS = jnp.sum(jnp.exp(s - mS), axis=1, keepdims=True)

    rowsum = num / seT - mT - jnp.log(seT) + mS + jnp.log(seS)
    total = jnp.sum(rowsum)

    @pl.when(pl.program_id(0) == 0)
    def _():
        out_ref[0, 0] = 0.0

    out_ref[0, 0] += total


def _tc_partial(preds_S, preds_T):
    return pl.pallas_call(
        _kl_block,
        grid=(NT // BLOCK,),
        in_specs=[
            pl.BlockSpec((BLOCK, C), lambda i: (i, 0)),
            pl.BlockSpec((BLOCK, C), lambda i: (i, 0)),
        ],
        out_specs=pl.BlockSpec(memory_space=pltpu.SMEM),
        out_shape=jax.ShapeDtypeStruct((1, 1), jnp.float32),
    )(preds_S, preds_T)


def _finish_block(p_ref, a_ref, b_ref, out_ref):
    out_ref[0, 0] = (p_ref[0, 0]
                     + jnp.sum(a_ref[...] + jnp.log(b_ref[...]))) * (1.0 / N)


def _finish(partial, a, b):
    out = pl.pallas_call(
        _finish_block,
        in_specs=[
            pl.BlockSpec(memory_space=pltpu.SMEM),
            pl.BlockSpec((SC_ROWS // 128, 128), lambda: (0, 0)),
            pl.BlockSpec((SC_ROWS // 128, 128), lambda: (0, 0)),
        ],
        out_specs=pl.BlockSpec(memory_space=pltpu.SMEM),
        out_shape=jax.ShapeDtypeStruct((1, 1), jnp.float32),
    )(partial, a.reshape(SC_ROWS // 128, 128), b.reshape(SC_ROWS // 128, 128))
    return out[0, 0]


@jax.jit
def kernel(preds_S, preds_T):
    a, b = _sc_partials(preds_S, preds_T)
    partial = _tc_partial(preds_S, preds_T)
    return _finish(partial, a, b)


# speedup vs baseline: 1.2628x; 1.2628x over previous
import jax
import jax.numpy as jnp
from jax.experimental import pallas as pl
from jax.experimental.pallas import tpu as pltpu

N, C = 16384, 1000
BLOCK = 1024


def _kl_block(s_ref, t_ref, out_ref):
    t = t_ref[...]
    s = s_ref[...]

    m1 = jnp.max(t, axis=1, keepdims=True)
    eq = t == m1
    dup = jnp.sum(eq.astype(jnp.float32), axis=1, keepdims=True) > 1.5
    m2 = jnp.max(jnp.where(eq, -jnp.inf, t), axis=1, keepdims=True)
    gap = m1 - m2
    sharp = (gap > 0.6) & (gap <= 0.8) & jnp.logical_not(dup)
    scale = jnp.where(sharp, 1.0 / 0.7, 1.0)

    ts = t * scale
    mT = m1 * scale
    eT = jnp.exp(ts - mT)
    seT = jnp.sum(eT, axis=1, keepdims=True)
    num = jnp.sum(eT * (ts - s), axis=1, keepdims=True)

    mS = jnp.max(s, axis=1, keepdims=True)
    seS = jnp.sum(jnp.exp(s - mS), axis=1, keepdims=True)

    rowsum = num / seT - mT - jnp.log(seT) + mS + jnp.log(seS)
    total = jnp.sum(rowsum) * (1.0 / N)

    @pl.when(pl.program_id(0) == 0)
    def _():
        out_ref[0, 0] = 0.0

    out_ref[0, 0] += total


@jax.jit
def kernel(preds_S, preds_T):
    out = pl.pallas_call(
        _kl_block,
        grid=(N // BLOCK,),
        in_specs=[
            pl.BlockSpec((BLOCK, C), lambda i: (i, 0)),
            pl.BlockSpec((BLOCK, C), lambda i: (i, 0)),
        ],
        out_specs=pl.BlockSpec(memory_space=pltpu.SMEM),
        out_shape=jax.ShapeDtypeStruct((1, 1), jnp.float32),
    )(preds_S, preds_T)
    return out[0, 0]
